# Initial kernel scaffold; baseline (speedup 1.0000x reference)
import jax
import jax.numpy as jnp
from jax.experimental import pallas as pl


def kernel(x, edge_index, W, bias):
    raise NotImplementedError("write your pallas kernel here")


# speedup vs baseline: 5.4952x; 5.4952x over previous
import functools

import jax
import jax.numpy as jnp
from jax import lax
from jax.experimental import pallas as pl
from jax.experimental.pallas import tpu as pltpu
from jax.experimental.pallas import tpu_sc as plsc

B, C_IN, C_OUT, N, K = 1, 128, 128, 10000, 32
DEG = K + 1

NW = 32
NSUB = 16
NPT = 320
N_PAD = NW * NPT
GROUP = 4
GROUPS = NPT // GROUP
TC_BLK = 1024


def _tc_body(x_ref, w_ref, b_ref, o_ref):
    acc = lax.dot_general(
        x_ref[:, :], w_ref[:, :],
        (((0,), (1,)), ((), ())),
        preferred_element_type=jnp.float32,
    )
    o_ref[:, :] = jnp.maximum(acc, 0.0) + b_ref[:, :]


def _compute_h(x_pad, w_scaled, b_scaled):
    return pl.pallas_call(
        _tc_body,
        grid=(N_PAD // TC_BLK,),
        in_specs=[
            pl.BlockSpec((C_IN, TC_BLK), lambda i: (0, i)),
            pl.BlockSpec((C_OUT, C_IN), lambda i: (0, 0)),
            pl.BlockSpec((1, C_OUT), lambda i: (0, 0)),
        ],
        out_specs=pl.BlockSpec((TC_BLK, C_OUT), lambda i: (i, 0)),
        out_shape=jax.ShapeDtypeStruct((N_PAD, C_OUT), jnp.float32),
    )(x_pad, w_scaled, b_scaled)


def _sc_body(h_hbm, ei_hbm, dst_hbm, out_hbm,
             idx_v, dst_v, acc_v, rows_v, acc_sh, sem):
    cid = lax.axis_index("c")
    sid = lax.axis_index("s")
    wid = sid * 2 + cid
    node_base = wid * NPT
    slot_base = sid * NPT

    pltpu.sync_copy(ei_hbm.at[wid], idx_v)
    pltpu.sync_copy(dst_hbm.at[sid], dst_v)
    pltpu.sync_copy(h_hbm.at[pl.ds(node_base, NPT)], acc_v)
    pltpu.sync_copy(acc_v, acc_sh.at[pl.ds(slot_base, NPT)])

    def body(g, carry):
        pltpu.async_copy(h_hbm.at[idx_v.at[g]], rows_v, sem).wait()
        pltpu.sync_copy(rows_v, acc_sh.at[dst_v.at[g]], add=True)
        return carry

    lax.fori_loop(0, GROUPS, body, 0)

    pltpu.sync_copy(acc_sh.at[pl.ds(slot_base, NPT)], acc_v)
    pltpu.sync_copy(acc_v, out_hbm.at[pl.ds(node_base, NPT)])


@functools.partial(
    pl.kernel,
    out_type=jax.ShapeDtypeStruct((N_PAD, C_OUT), jnp.float32),
    mesh=plsc.VectorSubcoreMesh(core_axis_name="c", subcore_axis_name="s"),
    scratch_types=[
        pltpu.VMEM((GROUPS, GROUP * K), jnp.int32),
        pltpu.VMEM((GROUPS, GROUP * K), jnp.int32),
        pltpu.VMEM((NPT, C_OUT), jnp.float32),
        pltpu.VMEM((GROUP * K, C_OUT), jnp.float32),
        pltpu.VMEM_SHARED((NSUB * NPT, C_OUT), jnp.float32),
        pltpu.SemaphoreType.DMA,
    ],
)
def _sc_aggregate(h_hbm, ei_hbm, dst_hbm, out_hbm, *scratch):
    _sc_body(h_hbm, ei_hbm, dst_hbm, out_hbm, *scratch)


def kernel(x, edge_index, W, bias):
    x2 = x[0, :, :, 0]
    x_pad = jnp.pad(x2, ((0, 0), (0, N_PAD - N)))
    w_scaled = W * jnp.float32(1.0 / DEG)
    b_scaled = (bias[0, :, 0, 0] * jnp.float32(1.0 / DEG)).reshape(1, C_OUT)

    h = _compute_h(x_pad, w_scaled, b_scaled)

    ei = edge_index[0, 0].astype(jnp.int32)
    ei_pad = jnp.pad(ei, ((0, N_PAD - N), (0, 0)))
    ei_tiles = ei_pad.reshape(NW, GROUPS, GROUP * K)

    local = jnp.repeat(jnp.arange(NPT, dtype=jnp.int32), K).reshape(
        1, GROUPS, GROUP * K)
    dst_table = local + (jnp.arange(NSUB, dtype=jnp.int32) * NPT).reshape(
        NSUB, 1, 1)

    out_pad = _sc_aggregate(h, ei_tiles, dst_table)

    out = out_pad[:N].T
    return out.reshape(1, C_OUT, N, 1)


# speedup vs baseline: 5.7058x; 1.0383x over previous
import functools

import jax
import jax.numpy as jnp
from jax import lax
from jax.experimental import pallas as pl
from jax.experimental.pallas import tpu as pltpu
from jax.experimental.pallas import tpu_sc as plsc

B, C_IN, C_OUT, N, K = 1, 128, 128, 10000, 32
DEG = K + 1

NW = 32
NSUB = 16
NPT = 320
N_PAD = NW * NPT
GROUP = 4
GROUPS = NPT // GROUP
TC_BLK = 1024


def _tc_body(x_ref, w_ref, b_ref, o_ref):
    acc = lax.dot_general(
        x_ref[:, :], w_ref[:, :],
        (((0,), (1,)), ((), ())),
        preferred_element_type=jnp.float32,
    )
    o_ref[:, :] = jnp.maximum(acc, 0.0) + b_ref[:, :]


def _compute_h(x_pad, w_scaled, b_scaled):
    return pl.pallas_call(
        _tc_body,
        grid=(N_PAD // TC_BLK,),
        in_specs=[
            pl.BlockSpec((C_IN, TC_BLK), lambda i: (0, i)),
            pl.BlockSpec((C_OUT, C_IN), lambda i: (0, 0)),
            pl.BlockSpec((1, C_OUT), lambda i: (0, 0)),
        ],
        out_specs=pl.BlockSpec((TC_BLK, C_OUT), lambda i: (i, 0)),
        out_shape=jax.ShapeDtypeStruct((N_PAD, C_OUT), jnp.float32),
    )(x_pad, w_scaled, b_scaled)


def _sc_body(h_hbm, ei_hbm, dst_hbm, out_hbm,
             idx_v, dst_v, rows0_v, rows1_v, acc_sh,
             gsem0, gsem1, ssem0, ssem1):
    cid = lax.axis_index("c")
    sid = lax.axis_index("s")
    wid = sid * 2 + cid
    node_base = wid * NPT
    slot_base = sid * NPT

    pltpu.sync_copy(ei_hbm.at[wid], idx_v)
    pltpu.sync_copy(dst_hbm.at[sid], dst_v)
    pltpu.sync_copy(h_hbm.at[pl.ds(node_base, NPT)],
                    acc_sh.at[pl.ds(slot_base, NPT)])

    def body(i, carry):
        g0 = i * 2
        g1 = g0 + 1
        d0 = pltpu.async_copy(h_hbm.at[idx_v.at[g0]], rows0_v, gsem0)
        d1 = pltpu.async_copy(h_hbm.at[idx_v.at[g1]], rows1_v, gsem1)
        d0.wait()
        s0 = pltpu.async_copy(rows0_v, acc_sh.at[dst_v.at[g0]], ssem0,
                              add=True)
        d1.wait()
        s1 = pltpu.async_copy(rows1_v, acc_sh.at[dst_v.at[g1]], ssem1,
                              add=True)
        s0.wait()
        s1.wait()
        return carry

    lax.fori_loop(0, GROUPS // 2, body, 0)

    pltpu.sync_copy(acc_sh.at[pl.ds(slot_base, NPT)],
                    out_hbm.at[pl.ds(node_base, NPT)])


@functools.partial(
    pl.kernel,
    out_type=jax.ShapeDtypeStruct((N_PAD, C_OUT), jnp.float32),
    mesh=plsc.VectorSubcoreMesh(core_axis_name="c", subcore_axis_name="s"),
    scratch_types=[
        pltpu.VMEM((GROUPS, GROUP * K), jnp.int32),
        pltpu.VMEM((GROUPS, GROUP * K), jnp.int32),
        pltpu.VMEM((GROUP * K, C_OUT), jnp.float32),
        pltpu.VMEM((GROUP * K, C_OUT), jnp.float32),
        pltpu.VMEM_SHARED((NSUB * NPT, C_OUT), jnp.float32),
        pltpu.SemaphoreType.DMA,
        pltpu.SemaphoreType.DMA,
        pltpu.SemaphoreType.DMA,
        pltpu.SemaphoreType.DMA,
    ],
)
def _sc_aggregate(h_hbm, ei_hbm, dst_hbm, out_hbm, *scratch):
    _sc_body(h_hbm, ei_hbm, dst_hbm, out_hbm, *scratch)


def kernel(x, edge_index, W, bias):
    x2 = x[0, :, :, 0]
    x_pad = jnp.pad(x2, ((0, 0), (0, N_PAD - N)))
    w_scaled = W * jnp.float32(1.0 / DEG)
    b_scaled = (bias[0, :, 0, 0] * jnp.float32(1.0 / DEG)).reshape(1, C_OUT)

    h = _compute_h(x_pad, w_scaled, b_scaled)

    ei = edge_index[0, 0].astype(jnp.int32)
    ei_pad = jnp.pad(ei, ((0, N_PAD - N), (0, 0)))
    ei_tiles = ei_pad.reshape(NW, GROUPS, GROUP * K)

    local = jnp.repeat(jnp.arange(NPT, dtype=jnp.int32), K).reshape(
        1, GROUPS, GROUP * K)
    dst_table = local + (jnp.arange(NSUB, dtype=jnp.int32) * NPT).reshape(
        NSUB, 1, 1)

    out_pad = _sc_aggregate(h, ei_tiles, dst_table)

    out = out_pad[:N].T
    return out.reshape(1, C_OUT, N, 1)


# speedup vs baseline: 5.7212x; 1.0027x over previous
import functools

import jax
import jax.numpy as jnp
from jax import lax
from jax.experimental import pallas as pl
from jax.experimental.pallas import tpu as pltpu
from jax.experimental.pallas import tpu_sc as plsc

B, C_IN, C_OUT, N, K = 1, 128, 128, 10000, 32
DEG = K + 1

NW = 32
NSUB = 16
NPT = 320
N_PAD = NW * NPT
GROUP = 4
GROUPS = NPT // GROUP
NBUF = 4
TC_BLK = 1024


def _tc_body(x_ref, w_ref, b_ref, o_ref):
    acc = lax.dot_general(
        x_ref[:, :], w_ref[:, :],
        (((0,), (1,)), ((), ())),
        preferred_element_type=jnp.float32,
    )
    o_ref[:, :] = jnp.maximum(acc, 0.0) + b_ref[:, :]


def _compute_h(x_pad, w_scaled, b_scaled):
    return pl.pallas_call(
        _tc_body,
        grid=(N_PAD // TC_BLK,),
        in_specs=[
            pl.BlockSpec((C_IN, TC_BLK), lambda i: (0, i)),
            pl.BlockSpec((C_OUT, C_IN), lambda i: (0, 0)),
            pl.BlockSpec((1, C_OUT), lambda i: (0, 0)),
        ],
        out_specs=pl.BlockSpec((TC_BLK, C_OUT), lambda i: (i, 0)),
        out_shape=jax.ShapeDtypeStruct((N_PAD, C_OUT), jnp.float32),
    )(x_pad, w_scaled, b_scaled)


def _sc_body(h_hbm, ei_hbm, dst_hbm, out_hbm,
             idx_v, dst_v, rows0_v, rows1_v, rows2_v, rows3_v, acc_sh,
             gsem0, gsem1, gsem2, gsem3, ssem0, ssem1, ssem2, ssem3):
    cid = lax.axis_index("c")
    sid = lax.axis_index("s")
    wid = sid * 2 + cid
    node_base = wid * NPT
    slot_base = sid * NPT

    pltpu.sync_copy(ei_hbm.at[wid], idx_v)
    pltpu.sync_copy(dst_hbm.at[sid], dst_v)
    pltpu.sync_copy(h_hbm.at[pl.ds(node_base, NPT)],
                    acc_sh.at[pl.ds(slot_base, NPT)])

    rows = (rows0_v, rows1_v, rows2_v, rows3_v)
    gsems = (gsem0, gsem1, gsem2, gsem3)
    ssems = (ssem0, ssem1, ssem2, ssem3)

    def gather(g, b):
        pltpu.async_copy(h_hbm.at[idx_v.at[g]], rows[b], gsems[b])

    def wait_gather(g, b):
        pltpu.make_async_copy(h_hbm.at[idx_v.at[g]], rows[b],
                              gsems[b]).wait()

    def scatter(g, b):
        pltpu.async_copy(rows[b], acc_sh.at[dst_v.at[g]], ssems[b],
                         add=True)

    def wait_scatter(g, b):
        pltpu.make_async_copy(rows[b], acc_sh.at[dst_v.at[g]],
                              ssems[b]).wait()

    for b in range(NBUF):
        gather(b, b)

    def body(o, carry):
        g0 = o * NBUF
        for b in range(NBUF):
            wait_gather(g0 + b, b)
        for b in range(NBUF):
            scatter(g0 + b, b)
            wait_scatter(g0 + b, b)
        for b in range(NBUF):
            gather(g0 + NBUF + b, b)
        return carry

    lax.fori_loop(0, GROUPS // NBUF - 1, body, 0)

    g0 = GROUPS - NBUF
    for b in range(NBUF):
        wait_gather(g0 + b, b)
    for b in range(NBUF):
        scatter(g0 + b, b)
    for b in range(NBUF):
        wait_scatter(g0 + b, b)

    pltpu.sync_copy(acc_sh.at[pl.ds(slot_base, NPT)],
                    out_hbm.at[pl.ds(node_base, NPT)])


@functools.partial(
    pl.kernel,
    out_type=jax.ShapeDtypeStruct((N_PAD, C_OUT), jnp.float32),
    mesh=plsc.VectorSubcoreMesh(core_axis_name="c", subcore_axis_name="s"),
    scratch_types=[
        pltpu.VMEM((GROUPS, GROUP * K), jnp.int32),
        pltpu.VMEM((GROUPS, GROUP * K), jnp.int32),
        pltpu.VMEM((GROUP * K, C_OUT), jnp.float32),
        pltpu.VMEM((GROUP * K, C_OUT), jnp.float32),
        pltpu.VMEM((GROUP * K, C_OUT), jnp.float32),
        pltpu.VMEM((GROUP * K, C_OUT), jnp.float32),
        pltpu.VMEM_SHARED((NSUB * NPT, C_OUT), jnp.float32),
        pltpu.SemaphoreType.DMA,
        pltpu.SemaphoreType.DMA,
        pltpu.SemaphoreType.DMA,
        pltpu.SemaphoreType.DMA,
        pltpu.SemaphoreType.DMA,
        pltpu.SemaphoreType.DMA,
        pltpu.SemaphoreType.DMA,
        pltpu.SemaphoreType.DMA,
    ],
)
def _sc_aggregate(h_hbm, ei_hbm, dst_hbm, out_hbm, *scratch):
    _sc_body(h_hbm, ei_hbm, dst_hbm, out_hbm, *scratch)


def kernel(x, edge_index, W, bias):
    x2 = x[0, :, :, 0]
    x_pad = jnp.pad(x2, ((0, 0), (0, N_PAD - N)))
    w_scaled = W * jnp.float32(1.0 / DEG)
    b_scaled = (bias[0, :, 0, 0] * jnp.float32(1.0 / DEG)).reshape(1, C_OUT)

    h = _compute_h(x_pad, w_scaled, b_scaled)

    ei = edge_index[0, 0].astype(jnp.int32)
    ei_pad = jnp.pad(ei, ((0, N_PAD - N), (0, 0)))
    ei_tiles = ei_pad.reshape(NW, GROUPS, GROUP * K)

    local = jnp.repeat(jnp.arange(NPT, dtype=jnp.int32), K).reshape(
        1, GROUPS, GROUP * K)
    dst_table = local + (jnp.arange(NSUB, dtype=jnp.int32) * NPT).reshape(
        NSUB, 1, 1)

    out_pad = _sc_aggregate(h, ei_tiles, dst_table)

    out = out_pad[:N].T
    return out.reshape(1, C_OUT, N, 1)


# speedup vs baseline: 5.8118x; 1.0158x over previous
import functools

import jax
import jax.numpy as jnp
from jax import lax
from jax.experimental import pallas as pl
from jax.experimental.pallas import tpu as pltpu
from jax.experimental.pallas import tpu_sc as plsc

B, C_IN, C_OUT, N, K = 1, 128, 128, 10000, 32
DEG = K + 1

NW = 32
NPT = 320
N_PAD = NW * NPT
GROUP = 4
GROUPS = NPT // GROUP
NBUF = 4
LANES = 16
CVECS = C_OUT // LANES
TC_BLK = 1024


def _tc_body(x_ref, w_ref, b_ref, o_ref):
    acc = lax.dot_general(
        x_ref[:, :], w_ref[:, :],
        (((0,), (1,)), ((), ())),
        preferred_element_type=jnp.float32,
    )
    o_ref[:, :] = jnp.maximum(acc, 0.0) + b_ref[:, :]


def _compute_h(x_pad, w_scaled, b_scaled):
    return pl.pallas_call(
        _tc_body,
        grid=(N_PAD // TC_BLK,),
        in_specs=[
            pl.BlockSpec((C_IN, TC_BLK), lambda i: (0, i)),
            pl.BlockSpec((C_OUT, C_IN), lambda i: (0, 0)),
            pl.BlockSpec((1, C_OUT), lambda i: (0, 0)),
        ],
        out_specs=pl.BlockSpec((TC_BLK, C_OUT), lambda i: (i, 0)),
        out_shape=jax.ShapeDtypeStruct((N_PAD, C_OUT), jnp.float32),
    )(x_pad, w_scaled, b_scaled)


def _sc_body(h_hbm, ei_hbm, out_hbm,
             idx_v, acc_v, rows0_v, rows1_v, rows2_v, rows3_v,
             gsem0, gsem1, gsem2, gsem3):
    cid = lax.axis_index("c")
    sid = lax.axis_index("s")
    wid = sid * 2 + cid
    node_base = wid * NPT

    rows = (rows0_v, rows1_v, rows2_v, rows3_v)
    gsems = (gsem0, gsem1, gsem2, gsem3)

    pltpu.sync_copy(ei_hbm.at[wid], idx_v)
    pltpu.sync_copy(h_hbm.at[pl.ds(node_base, NPT)], acc_v)

    def gather(g, b):
        pltpu.async_copy(h_hbm.at[idx_v.at[g]], rows[b], gsems[b])

    def wait_gather(g, b):
        pltpu.make_async_copy(h_hbm.at[idx_v.at[g]], rows[b],
                              gsems[b]).wait()

    def consume(g, b):
        rows_b = rows[b]

        def node(n_, carry):
            row = g * GROUP + n_
            base = n_ * K
            for c in range(CVECS):
                cs = pl.ds(c * LANES, LANES)
                s = acc_v[row, cs]
                for r in range(K):
                    s = s + rows_b[base + r, cs]
                acc_v[row, cs] = s
            return carry

        lax.fori_loop(0, GROUP, node, 0)

    for b in range(NBUF):
        gather(b, b)

    def body(o, carry):
        g0 = o * NBUF
        for b in range(NBUF):
            g = g0 + b
            wait_gather(g, b)
            consume(g, b)
            gather(g + NBUF, b)
        return carry

    lax.fori_loop(0, GROUPS // NBUF - 1, body, 0)

    g0 = GROUPS - NBUF
    for b in range(NBUF):
        wait_gather(g0 + b, b)
        consume(g0 + b, b)

    pltpu.sync_copy(acc_v, out_hbm.at[pl.ds(node_base, NPT)])


@functools.partial(
    pl.kernel,
    out_type=jax.ShapeDtypeStruct((N_PAD, C_OUT), jnp.float32),
    mesh=plsc.VectorSubcoreMesh(core_axis_name="c", subcore_axis_name="s"),
    scratch_types=[
        pltpu.VMEM((GROUPS, GROUP * K), jnp.int32),
        pltpu.VMEM((NPT, C_OUT), jnp.float32),
        pltpu.VMEM((GROUP * K, C_OUT), jnp.float32),
        pltpu.VMEM((GROUP * K, C_OUT), jnp.float32),
        pltpu.VMEM((GROUP * K, C_OUT), jnp.float32),
        pltpu.VMEM((GROUP * K, C_OUT), jnp.float32),
        pltpu.SemaphoreType.DMA,
        pltpu.SemaphoreType.DMA,
        pltpu.SemaphoreType.DMA,
        pltpu.SemaphoreType.DMA,
    ],
)
def _sc_aggregate(h_hbm, ei_hbm, out_hbm, *scratch):
    _sc_body(h_hbm, ei_hbm, out_hbm, *scratch)


def kernel(x, edge_index, W, bias):
    x2 = x[0, :, :, 0]
    x_pad = jnp.pad(x2, ((0, 0), (0, N_PAD - N)))
    w_scaled = W * jnp.float32(1.0 / DEG)
    b_scaled = (bias[0, :, 0, 0] * jnp.float32(1.0 / DEG)).reshape(1, C_OUT)

    h = _compute_h(x_pad, w_scaled, b_scaled)

    ei = edge_index[0, 0].astype(jnp.int32)
    ei_pad = jnp.pad(ei, ((0, N_PAD - N), (0, 0)))
    ei_tiles = ei_pad.reshape(NW, GROUPS, GROUP * K)

    out_pad = _sc_aggregate(h, ei_tiles)

    out = out_pad[:N].T
    return out.reshape(1, C_OUT, N, 1)
